# baseline (speedup 1.0000x reference)
import functools

import jax
import jax.numpy as jnp
from jax import lax
from jax.experimental import pallas as pl
from jax.experimental.pallas import tpu as pltpu
from jax.experimental.pallas import tpu_sc as plsc

MARGIN_ = 1.0
NC, NS, L = 2, 16, 16
NW = NC * NS
PAIRS = 16384
PW = PAIRS // NW
STEP = 128
NSTEPS = PW // STEP
D = 64


def _lane_perm(x, idx):
    dnums = lax.GatherDimensionNumbers(
        offset_dims=(), collapsed_slice_dims=(0,), start_index_map=(0,))
    return lax.gather(x, idx[:, None], dnums, (1,),
                      mode=lax.GatherScatterMode.PROMISE_IN_BOUNDS)


def _sc_loss_kernel(ent_hbm, rel_hbm, ps_h, pr_h, pt_h, ns_h, nr_h, nt_h,
                    out_hbm,
                    ps_v, pr_v, pt_v, ns_v, nr_v, nt_v,
                    r_ps, r_pr, r_pt, r_ns, r_nr, r_nt,
                    acc_v, sem):
    wid = lax.axis_index("s") * NC + lax.axis_index("c")

    pltpu.sync_copy(ps_h.at[wid], ps_v)
    pltpu.sync_copy(pr_h.at[wid], pr_v)
    pltpu.sync_copy(pt_h.at[wid], pt_v)
    pltpu.sync_copy(ns_h.at[wid], ns_v)
    pltpu.sync_copy(nr_h.at[wid], nr_v)
    pltpu.sync_copy(nt_h.at[wid], nt_v)

    def step_body(j, loss):
        cps = pltpu.async_copy(ent_hbm.at[ps_v.at[j]], r_ps, sem)
        cpr = pltpu.async_copy(rel_hbm.at[pr_v.at[j]], r_pr, sem)
        cpt = pltpu.async_copy(ent_hbm.at[pt_v.at[j]], r_pt, sem)
        cns = pltpu.async_copy(ent_hbm.at[ns_v.at[j]], r_ns, sem)
        cnr = pltpu.async_copy(rel_hbm.at[nr_v.at[j]], r_nr, sem)
        cnt = pltpu.async_copy(ent_hbm.at[nt_v.at[j]], r_nt, sem)
        for c in (cps, cpr, cpt, cns, cnr, cnt):
            c.wait()

        lanes = lax.iota(jnp.int32, L)

        def pair_block(pb, acc):
            for u in range(4):
                p = pb * 4 + u
                d = jnp.zeros((L,), jnp.float32)
                for k in range(D // L):
                    sl = pl.ds(k * L, L)
                    xp = jnp.abs(r_ps[p, sl] + r_pr[p, sl] - r_pt[p, sl])
                    xn = jnp.abs(r_ns[p, sl] + r_nr[p, sl] - r_nt[p, sl])
                    d = d + (xp - xn)
                for sh in (8, 4, 2, 1):
                    d = d + _lane_perm(d, lanes ^ sh)
                acc = acc + jnp.maximum(d + MARGIN_, 0.0)
            return acc

        return lax.fori_loop(0, STEP // 4, pair_block, loss)

    acc = lax.fori_loop(0, NSTEPS, step_body, jnp.zeros((L,), jnp.float32))

    lanes = lax.iota(jnp.int32, L)
    acc_v[...] = jnp.where(lanes == 0, acc, 0.0)
    pltpu.sync_copy(acc_v, out_hbm.at[wid])


@jax.jit
def kernel(train_indices, ent_embeds, rel_embeds):
    idx = train_indices.astype(jnp.int32)
    pos = idx[:PAIRS]
    neg = idx[PAIRS:]
    blocks = [c.reshape(NW, NSTEPS, STEP)
              for c in (pos[:, 0], pos[:, 1], pos[:, 2],
                        neg[:, 0], neg[:, 1], neg[:, 2])]

    mesh = plsc.VectorSubcoreMesh(core_axis_name="c", subcore_axis_name="s")
    run = functools.partial(
        pl.kernel,
        mesh=mesh,
        compiler_params=pltpu.CompilerParams(use_tc_tiling_on_sc=False),
        out_type=jax.ShapeDtypeStruct((NW, L), jnp.float32),
        scratch_types=(
            [pltpu.VMEM((NSTEPS, STEP), jnp.int32)] * 6
            + [pltpu.VMEM((STEP, D), jnp.float32)] * 6
            + [pltpu.VMEM((L,), jnp.float32), pltpu.SemaphoreType.DMA]
        ),
    )(_sc_loss_kernel)
    partials = run(ent_embeds, rel_embeds, *blocks)
    return jnp.sum(partials) / PAIRS
